# baseline (speedup 1.0000x reference)
import functools

import jax
import jax.numpy as jnp
from jax import lax
from jax.experimental import pallas as pl
from jax.experimental.pallas import tpu as pltpu
from jax.experimental.pallas import tpu_sc as plsc

_NUM_CORES = 2
_NUM_SUBCORES = 16
_NW = _NUM_CORES * _NUM_SUBCORES
_D = 32
_LANES = 16
_TILE_B = 128
_HPC = 1
_PW = 33


@functools.cache
def _make_gather(batch: int, hist: int):
    B = batch * hist
    bw = batch // _NW
    nbt = bw // _TILE_B
    ncg = _D // 8
    t1_len = bw * _D
    slab = _D * batch
    chunk = _HPC * bw
    n_chunks = hist // _HPC
    mesh = plsc.VectorSubcoreMesh(core_axis_name="c", subcore_axis_name="s")

    @functools.partial(
        pl.kernel,
        out_type=jax.ShapeDtypeStruct((B * _D,), jnp.float32),
        mesh=mesh,
        scratch_types=[
            pltpu.VMEM((hist * bw,), jnp.int32),
            pltpu.VMEM((chunk, _D), jnp.float32),
            pltpu.VMEM((chunk, _D), jnp.float32),
            pltpu.VMEM((chunk, _D), jnp.float32),
            pltpu.VMEM((chunk * _PW,), jnp.float32),
            pltpu.VMEM((t1_len,), jnp.float32),
            pltpu.VMEM((t1_len,), jnp.float32),
            pltpu.SemaphoreType.DMA,
            pltpu.SemaphoreType.DMA,
            pltpu.SemaphoreType.DMA,
            pltpu.SemaphoreType.DMA,
            pltpu.SemaphoreType.DMA,
            pltpu.SemaphoreType.DMA,
        ],
        compiler_params=pltpu.CompilerParams(
            use_tc_tiling_on_sc=False, needs_layout_passes=False
        ),
    )
    def gather_kernel(
        table_hbm, idx_hbm, out_hbm,
        idx_v, r0, r1, r2, pstage, t0, t1,
        g0, g1, g2, isem, s0, s1,
    ):
        wid = lax.axis_index("s") * _NUM_CORES + lax.axis_index("c")
        col0 = wid * bw
        rows = (r0, r1, r2)
        gsem = (g0, g1, g2)
        tiles = (t0, t1)
        ssem = (s0, s1)

        for h in range(hist):
            pltpu.async_copy(
                idx_hbm.at[pl.ds(h * batch + col0, bw)],
                idx_v.at[pl.ds(h * bw, bw)],
                isem,
            )
        for h in range(hist):
            pltpu.make_async_copy(
                idx_hbm.at[pl.ds(0, bw)], idx_v.at[pl.ds(0, bw)], isem
            ).wait()

        lane = lax.iota(jnp.int32, _LANES)
        cvec0 = lane * 128
        cvec1 = cvec0 + 2048

        def fire_gather(g, rb):
            pltpu.async_copy(
                table_hbm.at[idx_v.at[pl.ds(g * chunk, chunk)]], rows[rb], gsem[rb]
            )

        def wait_gather(rb):
            pltpu.make_async_copy(
                table_hbm.at[idx_v.at[pl.ds(0, chunk)]], rows[rb], gsem[rb]
            ).wait()

        def transpose(rb, tb):
            src = rows[rb]
            dst = tiles[tb]

            def widen_body(i, carry):
                for j in range(8):
                    b = i * 8 + j
                    v0 = src[b, pl.ds(0, _LANES)]
                    v1 = src[b, pl.ds(_LANES, _LANES)]
                    plsc.store_scatter(pstage, [lane + b * _PW], v0)
                    plsc.store_scatter(pstage, [lane + (b * _PW + _LANES)], v1)
                return carry

            lax.fori_loop(0, bw // 8, widen_body, 0)

            def col_body(i, carry):
                for cj in range(4):
                    c = i * 4 + cj

                    def blk(k, carry2):
                        b0 = k * _LANES
                        vals = plsc.load_gather(
                            pstage, [(b0 + lane) * _PW + c]
                        )
                        boff = (b0 >> 7) * (ncg * 8 * 128) + (b0 & 127)
                        coff = (c >> 3) * 1024 + (c & 7) * 128
                        plsc.store_scatter(dst, [lane + (coff + boff)], vals)
                        return carry2

                    lax.fori_loop(0, bw // _LANES, blk, 0)
                return carry

            lax.fori_loop(0, _D // 4, col_body, 0)

        def fire_stores(h, tb):
            for bt in range(nbt):
                for cg in range(ncg):
                    pltpu.async_copy(
                        tiles[tb].at[pl.ds((bt * ncg + cg) * 1024, 1024)],
                        out_hbm.at[
                            pl.ds(
                                h * slab + cg * (batch * 8)
                                + (wid * nbt + bt) * 1024,
                                1024,
                            )
                        ],
                        ssem[tb],
                    )

        def drain_stores(tb):
            for _ in range(nbt * ncg):
                pltpu.make_async_copy(
                    tiles[tb].at[pl.ds(0, 1024)],
                    out_hbm.at[pl.ds(0, 1024)],
                    ssem[tb],
                ).wait()

        for p in range(3):
            fire_gather(p, p)

        def step(g, rb, tb, drain, fire):
            wait_gather(rb)
            if drain:
                drain_stores(tb)
            transpose(rb, tb)
            fire_stores(g, tb)
            if fire == "always":
                fire_gather(g + 3, rb)
            elif fire == "cond":
                nxt = g + 3

                @pl.when(nxt < n_chunks)
                def _():
                    fire_gather(nxt, rb)

        for k in range(6):
            step(k, k % 3, k % 2, drain=k >= 2, fire="always")

        def body(i2, carry):
            for k in range(6):
                g = 6 + 6 * i2 + k
                step(g, k % 3, k % 2, drain=True, fire="cond")
            return carry

        n_main = (n_chunks - 6 - 2) // 6
        lax.fori_loop(0, n_main, body, 0)

        for k in range(n_chunks - 6 - n_main * 6):
            g = 6 + n_main * 6 + k
            step(g, g % 3, g % 2, drain=True, fire="never")

        for tb in range(2):
            drain_stores(tb)

    return gather_kernel


def kernel(x, table):
    batch, hist = x.shape
    idx = x.T.reshape(batch * hist).astype(jnp.int32)
    flat = _make_gather(batch, hist)(table, idx)
    a = flat.reshape(hist, _D // 8, batch // _TILE_B, 8, _TILE_B)
    return a.transpose(2, 4, 0, 1, 3).reshape(batch, hist, _D)


# speedup vs baseline: 1.0076x; 1.0076x over previous
import functools

import jax
import jax.numpy as jnp
from jax import lax
from jax.experimental import pallas as pl
from jax.experimental.pallas import tpu as pltpu
from jax.experimental.pallas import tpu_sc as plsc

_NUM_CORES = 2
_NUM_SUBCORES = 16
_NW = _NUM_CORES * _NUM_SUBCORES
_D = 32
_LANES = 16
_TILE_B = 128
_HPC = 1
_PW = 33


@functools.cache
def _make_gather(batch: int, hist: int):
    B = batch * hist
    bw = batch // _NW
    nbt = bw // _TILE_B
    ncg = _D // 8
    t1_len = bw * _D
    slab = _D * batch
    chunk = _HPC * bw
    n_chunks = hist // _HPC
    mesh = plsc.VectorSubcoreMesh(core_axis_name="c", subcore_axis_name="s")

    @functools.partial(
        pl.kernel,
        out_type=jax.ShapeDtypeStruct((B * _D,), jnp.float32),
        mesh=mesh,
        scratch_types=[
            pltpu.VMEM((hist * bw,), jnp.int32),
            pltpu.VMEM((chunk, _D), jnp.float32),
            pltpu.VMEM((chunk, _D), jnp.float32),
            pltpu.VMEM((chunk * _PW,), jnp.float32),
            pltpu.VMEM((t1_len,), jnp.float32),
            pltpu.VMEM((t1_len,), jnp.float32),
            pltpu.SemaphoreType.DMA,
            pltpu.SemaphoreType.DMA,
            pltpu.SemaphoreType.DMA,
            pltpu.SemaphoreType.DMA,
            pltpu.SemaphoreType.DMA,
        ],
        compiler_params=pltpu.CompilerParams(
            use_tc_tiling_on_sc=False, needs_layout_passes=False
        ),
    )
    def gather_kernel(
        table_hbm, idx_hbm, out_hbm,
        idx_v, r0, r1, pstage, t0, t1,
        g0, g1, isem, s0, s1,
    ):
        wid = lax.axis_index("s") * _NUM_CORES + lax.axis_index("c")
        col0 = wid * bw
        rows = (r0, r1)
        gsem = (g0, g1)
        tiles = (t0, t1)
        ssem = (s0, s1)

        for h in range(hist):
            pltpu.async_copy(
                idx_hbm.at[pl.ds(h * batch + col0, bw)],
                idx_v.at[pl.ds(h * bw, bw)],
                isem,
            )
        for h in range(hist):
            pltpu.make_async_copy(
                idx_hbm.at[pl.ds(0, bw)], idx_v.at[pl.ds(0, bw)], isem
            ).wait()

        lane = lax.iota(jnp.int32, _LANES)
        cvec0 = lane * 128
        cvec1 = cvec0 + 2048

        def fire_gather(g, rb):
            pltpu.async_copy(
                table_hbm.at[idx_v.at[pl.ds(g * chunk, chunk)]], rows[rb], gsem[rb]
            )

        def wait_gather(rb):
            pltpu.make_async_copy(
                table_hbm.at[idx_v.at[pl.ds(0, chunk)]], rows[rb], gsem[rb]
            ).wait()

        def transpose(rb, tb):
            src = rows[rb]
            dst = tiles[tb]

            def widen_body(i, carry):
                for j in range(8):
                    b = i * 8 + j
                    v0 = src[b, pl.ds(0, _LANES)]
                    v1 = src[b, pl.ds(_LANES, _LANES)]
                    plsc.store_scatter(pstage, [lane + b * _PW], v0)
                    plsc.store_scatter(pstage, [lane + (b * _PW + _LANES)], v1)
                return carry

            lax.fori_loop(0, bw // 8, widen_body, 0)

            def col_body(i, carry):
                for cj in range(4):
                    c = i * 4 + cj

                    def blk(k, carry2):
                        b0 = k * _LANES
                        vals = plsc.load_gather(
                            pstage, [(b0 + lane) * _PW + c]
                        )
                        boff = (b0 >> 7) * (ncg * 8 * 128) + (b0 & 127)
                        coff = (c >> 3) * 1024 + (c & 7) * 128
                        plsc.store_scatter(dst, [lane + (coff + boff)], vals)
                        return carry2

                    lax.fori_loop(0, bw // _LANES, blk, 0)
                return carry

            lax.fori_loop(0, _D // 4, col_body, 0)

        def fire_stores(h, tb):
            for bt in range(nbt):
                for cg in range(ncg):
                    pltpu.async_copy(
                        tiles[tb].at[pl.ds((bt * ncg + cg) * 1024, 1024)],
                        out_hbm.at[
                            pl.ds(
                                h * slab + cg * (batch * 8)
                                + (wid * nbt + bt) * 1024,
                                1024,
                            )
                        ],
                        ssem[tb],
                    )

        def drain_stores(tb):
            for _ in range(nbt * ncg):
                pltpu.make_async_copy(
                    tiles[tb].at[pl.ds(0, 1024)],
                    out_hbm.at[pl.ds(0, 1024)],
                    ssem[tb],
                ).wait()

        for p in range(2):
            fire_gather(p, p)

        def step(g, rb, tb, drain, fire):
            wait_gather(rb)
            if drain:
                drain_stores(tb)
            transpose(rb, tb)
            fire_stores(g, tb)
            if fire:
                fire_gather(g + 2, rb)

        step(0, 0, 0, drain=False, fire=True)
        step(1, 1, 1, drain=False, fire=True)

        def body(i2, carry):
            step(2 + 2 * i2, 0, 0, drain=True, fire=True)
            step(3 + 2 * i2, 1, 1, drain=True, fire=True)
            return carry

        n_main = (n_chunks - 4) // 2
        lax.fori_loop(0, n_main, body, 0)

        step(n_chunks - 2, 0, 0, drain=True, fire=False)
        step(n_chunks - 1, 1, 1, drain=True, fire=False)

        for tb in range(2):
            drain_stores(tb)

    return gather_kernel


def kernel(x, table):
    batch, hist = x.shape
    idx = x.T.reshape(batch * hist).astype(jnp.int32)
    flat = _make_gather(batch, hist)(table, idx)
    a = flat.reshape(hist, _D // 8, batch // _TILE_B, 8, _TILE_B)
    return a.transpose(2, 4, 0, 1, 3).reshape(batch, hist, _D)
